# Initial kernel scaffold; baseline (speedup 1.0000x reference)
import jax
import jax.numpy as jnp
from jax.experimental import pallas as pl


def kernel(x, edge_index, W1, b1, W2, b2, W3, b3, W4, b4, W5, b5, W_ih, W_hh, b_ih, b_hh, W_fc, b_fc):
    raise NotImplementedError("write your pallas kernel here")


# speedup vs baseline: 9.2981x; 9.2981x over previous
import functools

import jax
import jax.numpy as jnp
from jax import lax
from jax.experimental import pallas as pl
from jax.experimental.pallas import tpu as pltpu
from jax.experimental.pallas import tpu_sc as plsc

NC = 2
NS = 16
NW = NC * NS


@functools.partial(jax.jit, static_argnames=("n_pad",))
def _deg_sc(dst, zeros_rpt, ones_k, *, n_pad):
  E = dst.shape[0]
  EPC = E // NC
  EPT = EPC // NS
  K = ones_k.shape[0]
  STEPS = EPT // K
  RPT = n_pad // NS

  mesh = plsc.VectorSubcoreMesh(core_axis_name="c", subcore_axis_name="s")

  @functools.partial(
      pl.kernel,
      out_type=jax.ShapeDtypeStruct((NC * n_pad,), jnp.float32),
      mesh=mesh,
      scratch_types=[
          pltpu.VMEM((K,), jnp.int32),
          pltpu.VMEM((K,), jnp.float32),
          pltpu.VMEM((RPT,), jnp.float32),
          pltpu.VMEM_SHARED((n_pad,), jnp.float32),
      ],
  )
  def k(dst_hbm, zeros_hbm, ones_hbm, out_hbm, didx, ones_v, bounce, acc):
    c = lax.axis_index("c")
    s = lax.axis_index("s")
    pltpu.sync_copy(ones_hbm, ones_v)
    pltpu.sync_copy(zeros_hbm, bounce)
    pltpu.sync_copy(bounce, acc.at[pl.ds(s * RPT, RPT)])
    plsc.subcore_barrier()

    @pl.loop(0, STEPS)
    def body(j):
      off = c * EPC + s * EPT + j * K
      pltpu.sync_copy(dst_hbm.at[pl.ds(off, K)], didx)
      pltpu.sync_copy(ones_v, acc.at[didx], add=True)

    plsc.subcore_barrier()
    pltpu.sync_copy(acc.at[pl.ds(s * RPT, RPT)], bounce)
    pltpu.sync_copy(bounce, out_hbm.at[pl.ds(c * n_pad + s * RPT, RPT)])

  return k(dst, zeros_rpt, ones_k)


@functools.partial(jax.jit, static_argnames=("n_pad",))
def _segsum_sc(src, dst, table, zeros_ch, *, n_pad):
  E = src.shape[0]
  D = table.shape[1]
  EPC = E // NC
  EPT = EPC // NS
  K = 80
  STEPS = EPT // K
  RPT = n_pad // NS
  CH = zeros_ch.shape[0]
  NCHUNK = RPT // CH

  mesh = plsc.VectorSubcoreMesh(core_axis_name="c", subcore_axis_name="s")

  @functools.partial(
      pl.kernel,
      out_type=jax.ShapeDtypeStruct((NC * n_pad, D), jnp.float32),
      mesh=mesh,
      scratch_types=[
          pltpu.VMEM((K,), jnp.int32),
          pltpu.VMEM((K,), jnp.int32),
          pltpu.VMEM((K, 128), jnp.float32),
          pltpu.VMEM((128, 128), jnp.float32),
          pltpu.VMEM_SHARED((n_pad, 128), jnp.float32),
          pltpu.SemaphoreType.DMA,
      ],
  )
  def k(src_hbm, dst_hbm, tab_hbm, zeros_hbm, out_hbm,
        sidx, didx, rows, bounce, acc, sem):
    c = lax.axis_index("c")
    s = lax.axis_index("s")
    pltpu.sync_copy(zeros_hbm, bounce)

    @pl.loop(0, NCHUNK)
    def zacc(j):
      pltpu.sync_copy(bounce, acc.at[pl.ds(s * RPT + j * CH, CH)])

    plsc.subcore_barrier()

    @pl.loop(0, STEPS)
    def body(j):
      off = c * EPC + s * EPT + j * K
      pltpu.sync_copy(src_hbm.at[pl.ds(off, K)], sidx)
      pltpu.sync_copy(dst_hbm.at[pl.ds(off, K)], didx)
      pltpu.async_copy(tab_hbm.at[sidx], rows, sem).wait()
      pltpu.sync_copy(rows, acc.at[didx], add=True)

    plsc.subcore_barrier()

    @pl.loop(0, NCHUNK)
    def cout(j):
      r0 = s * RPT + j * CH
      pltpu.sync_copy(acc.at[pl.ds(r0, CH)], bounce)
      pltpu.sync_copy(bounce, out_hbm.at[pl.ds(c * n_pad + r0, CH)])

  return k(src, dst, table, zeros_ch)


_BR = 1000


def _tc_pre(x, W1, d0, d1):
  n = x.shape[0]

  def body(x_ref, w_ref, d0_ref, d1_ref, tp_ref, dv_ref):
    deg = d0_ref[...] + d1_ref[...] + 1.0
    dv = lax.rsqrt(deg)
    t = jnp.dot(x_ref[...], w_ref[...], preferred_element_type=jnp.float32)
    tp_ref[...] = t * dv
    dv_ref[...] = dv

  return pl.pallas_call(
      body,
      grid=(n // _BR,),
      in_specs=[
          pl.BlockSpec((_BR, 128), lambda i: (i, 0)),
          pl.BlockSpec((128, 128), lambda i: (0, 0)),
          pl.BlockSpec((_BR, 1), lambda i: (i, 0)),
          pl.BlockSpec((_BR, 1), lambda i: (i, 0)),
      ],
      out_specs=[
          pl.BlockSpec((_BR, 128), lambda i: (i, 0)),
          pl.BlockSpec((_BR, 1), lambda i: (i, 0)),
      ],
      out_shape=[
          jax.ShapeDtypeStruct((n, 128), jnp.float32),
          jax.ShapeDtypeStruct((n, 1), jnp.float32),
      ],
  )(x, W1, d0, d1)


def _tc_fuse(p0, p1, tp, dinv, b, Wnext):
  n = tp.shape[0]

  def body(p0_ref, p1_ref, tp_ref, dv_ref, b_ref, w_ref, o_ref):
    dv = dv_ref[...]
    h = jax.nn.relu((p0_ref[...] + p1_ref[...] + tp_ref[...]) * dv + b_ref[...])
    o_ref[...] = jnp.dot(h, w_ref[...], preferred_element_type=jnp.float32) * dv

  return pl.pallas_call(
      body,
      grid=(n // _BR,),
      in_specs=[
          pl.BlockSpec((_BR, 128), lambda i: (i, 0)),
          pl.BlockSpec((_BR, 128), lambda i: (i, 0)),
          pl.BlockSpec((_BR, 128), lambda i: (i, 0)),
          pl.BlockSpec((_BR, 1), lambda i: (i, 0)),
          pl.BlockSpec((1, 128), lambda i: (0, 0)),
          pl.BlockSpec((128, 128), lambda i: (0, 0)),
      ],
      out_specs=pl.BlockSpec((_BR, 128), lambda i: (i, 0)),
      out_shape=jax.ShapeDtypeStruct((n, 128), jnp.float32),
  )(p0, p1, tp, dinv, b, Wnext)


def _tc_final(p0, p1, tp, dinv, b5, Wg, bg, WfcT, bfc):
  n = tp.shape[0]
  H = 128

  def body(p0_ref, p1_ref, tp_ref, dv_ref, b5_ref, wg_ref, bg_ref,
           wfc_ref, bfc_ref, o_ref):
    dv = dv_ref[...]
    h = jax.nn.relu((p0_ref[...] + p1_ref[...] + tp_ref[...]) * dv + b5_ref[...])
    g3 = jnp.dot(h, wg_ref[...], preferred_element_type=jnp.float32) + bg_ref[...]
    i_g = jax.nn.sigmoid(g3[:, 0:H])
    g_g = jnp.tanh(g3[:, H:2 * H])
    o_g = jax.nn.sigmoid(g3[:, 2 * H:3 * H])
    hn = o_g * jnp.tanh(i_g * g_g)
    o_ref[...] = (jnp.dot(hn, wfc_ref[...], preferred_element_type=jnp.float32)
                  + bfc_ref[...])

  nc = WfcT.shape[1]
  return pl.pallas_call(
      body,
      grid=(n // _BR,),
      in_specs=[
          pl.BlockSpec((_BR, 128), lambda i: (i, 0)),
          pl.BlockSpec((_BR, 128), lambda i: (i, 0)),
          pl.BlockSpec((_BR, 128), lambda i: (i, 0)),
          pl.BlockSpec((_BR, 1), lambda i: (i, 0)),
          pl.BlockSpec((1, 128), lambda i: (0, 0)),
          pl.BlockSpec((128, 384), lambda i: (0, 0)),
          pl.BlockSpec((1, 384), lambda i: (0, 0)),
          pl.BlockSpec((128, nc), lambda i: (0, 0)),
          pl.BlockSpec((1, nc), lambda i: (0, 0)),
      ],
      out_specs=pl.BlockSpec((_BR, nc), lambda i: (i, 0)),
      out_shape=jax.ShapeDtypeStruct((n, nc), jnp.float32),
  )(p0, p1, tp, dinv, b5, Wg, bg, WfcT, bfc)


def kernel(x, edge_index, W1, b1, W2, b2, W3, b3, W4, b4, W5, b5,
           W_ih, W_hh, b_ih, b_hh, W_fc, b_fc):
  n = x.shape[0]
  H = 128
  n_pad = ((n + NW * 8 - 1) // (NW * 8)) * (NW * 8)

  src = edge_index[0]
  dst = edge_index[1]

  zeros_rpt = jnp.zeros((n_pad // NS,), jnp.float32)
  ones_k = jnp.ones((80,), jnp.float32)
  zeros_ch = jnp.zeros((128, 128), jnp.float32)

  degp = _deg_sc(dst, zeros_rpt, ones_k, n_pad=n_pad)
  d0 = degp[0:n, None]
  d1 = degp[n_pad:n_pad + n, None]

  tp, dinv = _tc_pre(x, W1, d0, d1)

  bs = [b1, b2, b3, b4]
  Ws = [W2, W3, W4, W5]
  for l in range(4):
    part = _segsum_sc(src, dst, tp, zeros_ch, n_pad=n_pad)
    p0 = part[0:n]
    p1 = part[n_pad:n_pad + n]
    tp = _tc_fuse(p0, p1, tp, dinv, bs[l].reshape(1, H), Ws[l])

  part = _segsum_sc(src, dst, tp, zeros_ch, n_pad=n_pad)
  p0 = part[0:n]
  p1 = part[n_pad:n_pad + n]

  Wg = jnp.concatenate([W_ih[0:H], W_ih[2 * H:4 * H]], axis=0).T
  bsum = (b_ih + b_hh)
  bg = jnp.concatenate([bsum[0:H], bsum[2 * H:4 * H]]).reshape(1, 3 * H)
  WfcT = W_fc.T
  bfc = b_fc.reshape(1, -1)

  return _tc_final(p0, p1, tp, dinv, b5.reshape(1, H), Wg, bg, WfcT, bfc)


# speedup vs baseline: 21.0790x; 2.2670x over previous
import functools

import jax
import jax.numpy as jnp
from jax import lax
from jax.experimental import pallas as pl
from jax.experimental.pallas import tpu as pltpu
from jax.experimental.pallas import tpu_sc as plsc

NC = 2
NS = 16
NW = NC * NS


@functools.partial(jax.jit, static_argnames=("n_pad",))
def _deg_sc(dst, zeros_rpt, ones_k, *, n_pad):
  E = dst.shape[0]
  EPC = E // NC
  EPT = EPC // NS
  K = ones_k.shape[0]
  STEPS = EPT // K
  RPT = n_pad // NS

  mesh = plsc.VectorSubcoreMesh(core_axis_name="c", subcore_axis_name="s")

  @functools.partial(
      pl.kernel,
      out_type=jax.ShapeDtypeStruct((NC * n_pad,), jnp.float32),
      mesh=mesh,
      scratch_types=[
          pltpu.VMEM((K,), jnp.int32),
          pltpu.VMEM((K,), jnp.float32),
          pltpu.VMEM((RPT,), jnp.float32),
          pltpu.VMEM_SHARED((n_pad,), jnp.float32),
      ],
  )
  def k(dst_hbm, zeros_hbm, ones_hbm, out_hbm, didx, ones_v, bounce, acc):
    c = lax.axis_index("c")
    s = lax.axis_index("s")
    pltpu.sync_copy(ones_hbm, ones_v)
    pltpu.sync_copy(zeros_hbm, bounce)
    pltpu.sync_copy(bounce, acc.at[pl.ds(s * RPT, RPT)])
    plsc.subcore_barrier()

    @pl.loop(0, STEPS)
    def body(j):
      off = c * EPC + s * EPT + j * K
      pltpu.sync_copy(dst_hbm.at[pl.ds(off, K)], didx)
      pltpu.sync_copy(ones_v, acc.at[didx], add=True)

    plsc.subcore_barrier()
    pltpu.sync_copy(acc.at[pl.ds(s * RPT, RPT)], bounce)
    pltpu.sync_copy(bounce, out_hbm.at[pl.ds(c * n_pad + s * RPT, RPT)])

  return k(dst, zeros_rpt, ones_k)


@functools.partial(jax.jit, static_argnames=("n_pad", "k_chunk"))
def _segsum_sc(src3d, dst3d, table, zeros_ch, *, n_pad, k_chunk):
  _, STEPS, K = src3d.shape
  D = table.shape[1]
  RPT = n_pad // NS

  mesh = plsc.VectorSubcoreMesh(core_axis_name="c", subcore_axis_name="s")

  @functools.partial(
      pl.kernel,
      out_type=jax.ShapeDtypeStruct((NC * n_pad, D), jnp.float32),
      mesh=mesh,
      scratch_types=[
          pltpu.VMEM((STEPS, K), jnp.int32),
          pltpu.VMEM((K,), jnp.int32),
          pltpu.VMEM((K,), jnp.int32),
          pltpu.VMEM((K, 128), jnp.float32),
          pltpu.VMEM((K, 128), jnp.float32),
          pltpu.VMEM_SHARED((n_pad, 128), jnp.float32),
          pltpu.SemaphoreType.DMA,
          pltpu.SemaphoreType.DMA,
          pltpu.SemaphoreType.DMA,
          pltpu.SemaphoreType.DMA,
          pltpu.SemaphoreType.DMA,
      ],
  )
  def k(src_hbm, dst_hbm, tab_hbm, zeros_hbm, out_hbm,
        sidx, didx_a, didx_b, rows_a, rows_b, acc,
        sem_i, sem_a, sem_b, sem_da, sem_db):
    c = lax.axis_index("c")
    s = lax.axis_index("s")
    wid = c * NS + s

    idx_s = pltpu.async_copy(src_hbm.at[wid], sidx, sem_i)
    pltpu.async_copy(dst_hbm.at[wid, 0], didx_a, sem_da)
    pltpu.async_copy(dst_hbm.at[wid, 1], didx_b, sem_db)
    pltpu.sync_copy(zeros_hbm, acc.at[pl.ds(s * RPT, RPT)])
    idx_s.wait()
    plsc.subcore_barrier()

    pltpu.async_copy(tab_hbm.at[sidx.at[0]], rows_a, sem_a)

    @pl.loop(0, STEPS, step=2)
    def pair(j):
      pltpu.make_async_copy(tab_hbm.at[sidx.at[j]], rows_a, sem_a).wait()
      pltpu.async_copy(tab_hbm.at[sidx.at[j + 1]], rows_b, sem_b)
      pltpu.make_async_copy(dst_hbm.at[wid, j], didx_a, sem_da).wait()
      pltpu.sync_copy(rows_a, acc.at[didx_a], add=True)

      @pl.when(j + 2 < STEPS)
      def _():
        pltpu.async_copy(tab_hbm.at[sidx.at[j + 2]], rows_a, sem_a)
        pltpu.async_copy(dst_hbm.at[wid, j + 2], didx_a, sem_da)

      pltpu.make_async_copy(tab_hbm.at[sidx.at[j + 1]], rows_b, sem_b).wait()
      pltpu.make_async_copy(dst_hbm.at[wid, j + 1], didx_b, sem_db).wait()
      pltpu.sync_copy(rows_b, acc.at[didx_b], add=True)

      @pl.when(j + 3 < STEPS)
      def _():
        pltpu.async_copy(dst_hbm.at[wid, j + 3], didx_b, sem_db)

    plsc.subcore_barrier()
    pltpu.sync_copy(acc.at[pl.ds(s * RPT, RPT)], out_hbm.at[pl.ds(c * n_pad + s * RPT, RPT)])

  return k(src3d, dst3d, table, zeros_ch)


_BR = 1000


def _tc_pre(x, W1, d0, d1):
  n = x.shape[0]

  def body(x_ref, w_ref, d0_ref, d1_ref, tp_ref, dv_ref):
    deg = d0_ref[...] + d1_ref[...] + 1.0
    dv = lax.rsqrt(deg)
    t = jnp.dot(x_ref[...], w_ref[...], preferred_element_type=jnp.float32)
    tp_ref[...] = t * dv
    dv_ref[...] = dv

  return pl.pallas_call(
      body,
      grid=(n // _BR,),
      in_specs=[
          pl.BlockSpec((_BR, 128), lambda i: (i, 0)),
          pl.BlockSpec((128, 128), lambda i: (0, 0)),
          pl.BlockSpec((_BR, 1), lambda i: (i, 0)),
          pl.BlockSpec((_BR, 1), lambda i: (i, 0)),
      ],
      out_specs=[
          pl.BlockSpec((_BR, 128), lambda i: (i, 0)),
          pl.BlockSpec((_BR, 1), lambda i: (i, 0)),
      ],
      out_shape=[
          jax.ShapeDtypeStruct((n, 128), jnp.float32),
          jax.ShapeDtypeStruct((n, 1), jnp.float32),
      ],
  )(x, W1, d0, d1)


def _tc_fuse(p0, p1, tp, dinv, b, Wnext):
  n = tp.shape[0]

  def body(p0_ref, p1_ref, tp_ref, dv_ref, b_ref, w_ref, o_ref):
    dv = dv_ref[...]
    h = jax.nn.relu((p0_ref[...] + p1_ref[...] + tp_ref[...]) * dv + b_ref[...])
    o_ref[...] = jnp.dot(h, w_ref[...], preferred_element_type=jnp.float32) * dv

  return pl.pallas_call(
      body,
      grid=(n // _BR,),
      in_specs=[
          pl.BlockSpec((_BR, 128), lambda i: (i, 0)),
          pl.BlockSpec((_BR, 128), lambda i: (i, 0)),
          pl.BlockSpec((_BR, 128), lambda i: (i, 0)),
          pl.BlockSpec((_BR, 1), lambda i: (i, 0)),
          pl.BlockSpec((1, 128), lambda i: (0, 0)),
          pl.BlockSpec((128, 128), lambda i: (0, 0)),
      ],
      out_specs=pl.BlockSpec((_BR, 128), lambda i: (i, 0)),
      out_shape=jax.ShapeDtypeStruct((n, 128), jnp.float32),
  )(p0, p1, tp, dinv, b, Wnext)


def _tc_final(p0, p1, tp, dinv, b5, Wg, bg, WfcT, bfc):
  n = tp.shape[0]
  H = 128

  def body(p0_ref, p1_ref, tp_ref, dv_ref, b5_ref, wg_ref, bg_ref,
           wfc_ref, bfc_ref, o_ref):
    dv = dv_ref[...]
    h = jax.nn.relu((p0_ref[...] + p1_ref[...] + tp_ref[...]) * dv + b5_ref[...])
    g3 = jnp.dot(h, wg_ref[...], preferred_element_type=jnp.float32) + bg_ref[...]
    i_g = jax.nn.sigmoid(g3[:, 0:H])
    g_g = jnp.tanh(g3[:, H:2 * H])
    o_g = jax.nn.sigmoid(g3[:, 2 * H:3 * H])
    hn = o_g * jnp.tanh(i_g * g_g)
    o_ref[...] = (jnp.dot(hn, wfc_ref[...], preferred_element_type=jnp.float32)
                  + bfc_ref[...])

  nc = WfcT.shape[1]
  return pl.pallas_call(
      body,
      grid=(n // _BR,),
      in_specs=[
          pl.BlockSpec((_BR, 128), lambda i: (i, 0)),
          pl.BlockSpec((_BR, 128), lambda i: (i, 0)),
          pl.BlockSpec((_BR, 128), lambda i: (i, 0)),
          pl.BlockSpec((_BR, 1), lambda i: (i, 0)),
          pl.BlockSpec((1, 128), lambda i: (0, 0)),
          pl.BlockSpec((128, 384), lambda i: (0, 0)),
          pl.BlockSpec((1, 384), lambda i: (0, 0)),
          pl.BlockSpec((128, nc), lambda i: (0, 0)),
          pl.BlockSpec((1, nc), lambda i: (0, 0)),
      ],
      out_specs=pl.BlockSpec((_BR, nc), lambda i: (i, 0)),
      out_shape=jax.ShapeDtypeStruct((n, nc), jnp.float32),
  )(p0, p1, tp, dinv, b5, Wg, bg, WfcT, bfc)


def kernel(x, edge_index, W1, b1, W2, b2, W3, b3, W4, b4, W5, b5,
           W_ih, W_hh, b_ih, b_hh, W_fc, b_fc):
  n = x.shape[0]
  H = 128
  n_pad = ((n + NW * 8 - 1) // (NW * 8)) * (NW * 8)

  src = edge_index[0]
  dst = edge_index[1]
  E = src.shape[0]
  K = 125
  STEPS = E // NW // K
  src3d = src.reshape(NW, STEPS, K)
  dst3d = dst.reshape(NW, STEPS, K)

  zeros_rpt = jnp.zeros((n_pad // NS,), jnp.float32)
  ones_k = jnp.ones((80,), jnp.float32)
  zeros_ch = jnp.zeros((n_pad // NS, 128), jnp.float32)

  degp = _deg_sc(dst, zeros_rpt, ones_k, n_pad=n_pad)
  d0 = degp[0:n, None]
  d1 = degp[n_pad:n_pad + n, None]

  tp, dinv = _tc_pre(x, W1, d0, d1)

  bs = [b1, b2, b3, b4]
  Ws = [W2, W3, W4, W5]
  for l in range(4):
    part = _segsum_sc(src3d, dst3d, tp, zeros_ch, n_pad=n_pad, k_chunk=K)
    p0 = part[0:n]
    p1 = part[n_pad:n_pad + n]
    tp = _tc_fuse(p0, p1, tp, dinv, bs[l].reshape(1, H), Ws[l])

  part = _segsum_sc(src3d, dst3d, tp, zeros_ch, n_pad=n_pad, k_chunk=K)
  p0 = part[0:n]
  p1 = part[n_pad:n_pad + n]

  Wg = jnp.concatenate([W_ih[0:H], W_ih[2 * H:4 * H]], axis=0).T
  bsum = (b_ih + b_hh)
  bg = jnp.concatenate([bsum[0:H], bsum[2 * H:4 * H]]).reshape(1, 3 * H)
  WfcT = W_fc.T
  bfc = b_fc.reshape(1, -1)

  return _tc_final(p0, p1, tp, dinv, b5.reshape(1, H), Wg, bg, WfcT, bfc)
